# Initial kernel scaffold; baseline (speedup 1.0000x reference)
import jax
import jax.numpy as jnp
from jax.experimental import pallas as pl


def kernel(x, edge_index, batch, enc_W, enc_b, gat_W, att_src, att_dst, gat_b, tag_W, tag_b, gc_W_rel, gc_b_rel, gc_W_root, gn1_w, gn1_b, gn1_a, gn2_w, gn2_b, gn2_a, cls_W1, cls_b1, cls_W2, cls_b2):
    raise NotImplementedError("write your pallas kernel here")


# speedup vs baseline: 1.0003x; 1.0003x over previous
import functools

import jax
import jax.numpy as jnp
from jax.experimental import pallas as pl
from jax.experimental.pallas import tpu as pltpu

N = 10000
E = 160000
G = 64
H = 256
HEADS = 4
HD = 64
K = 3
F = 7
C = 2

NB = 400


def _erf(x):
    s = jnp.sign(x)
    ax = jnp.abs(x)
    t = 1.0 / (1.0 + 0.3275911 * ax)
    poly = t * (0.254829592 + t * (-0.284496736 + t * (1.421413741
            + t * (-1.453152027 + t * 1.061405429))))
    return s * (1.0 - poly * jnp.exp(-ax * ax))


def _enc_body(x_ref, encW_ref, encb_ref, gatW_ref, atts_ref, attd_ref,
              h_ref, hp_ref, asrc_ref, adst_ref):
    x = x_ref[...]
    z = x @ encW_ref[...] + encb_ref[...]
    h = 0.5 * z * (1.0 + _erf(z * 0.7071067811865476))
    hp = h @ gatW_ref[...]
    h_ref[...] = h
    hp_ref[...] = hp
    asrc_ref[...] = hp @ atts_ref[...]
    adst_ref[...] = hp @ attd_ref[...]


def _encoder(x, enc_W, enc_b, gat_W, att_src, att_dst):
    eye = (jnp.arange(H)[:, None] // HD) == jnp.arange(HEADS)[None, :]
    attS = jnp.where(eye, att_src.reshape(H)[:, None], 0.0)
    attD = jnp.where(eye, att_dst.reshape(H)[:, None], 0.0)
    grid = (N // NB,)
    return pl.pallas_call(
        _enc_body,
        grid=grid,
        in_specs=[
            pl.BlockSpec((NB, F), lambda i: (i, 0)),
            pl.BlockSpec((F, H), lambda i: (0, 0)),
            pl.BlockSpec((H,), lambda i: (0,)),
            pl.BlockSpec((H, H), lambda i: (0, 0)),
            pl.BlockSpec((H, HEADS), lambda i: (0, 0)),
            pl.BlockSpec((H, HEADS), lambda i: (0, 0)),
        ],
        out_specs=[
            pl.BlockSpec((NB, H), lambda i: (i, 0)),
            pl.BlockSpec((NB, H), lambda i: (i, 0)),
            pl.BlockSpec((NB, HEADS), lambda i: (i, 0)),
            pl.BlockSpec((NB, HEADS), lambda i: (i, 0)),
        ],
        out_shape=[
            jax.ShapeDtypeStruct((N, H), jnp.float32),
            jax.ShapeDtypeStruct((N, H), jnp.float32),
            jax.ShapeDtypeStruct((N, HEADS), jnp.float32),
            jax.ShapeDtypeStruct((N, HEADS), jnp.float32),
        ],
    )(x, enc_W, enc_b, gat_W, attS, attD)


def _graph_norm(h, batch, w, b, a):
    cnt = jnp.maximum(jax.ops.segment_sum(jnp.ones((N,), jnp.float32), batch, num_segments=G), 1.0)
    mean = jax.ops.segment_sum(h, batch, num_segments=G) / cnt[:, None]
    out = h - a[None, :] * mean[batch]
    var = jax.ops.segment_sum(out * out, batch, num_segments=G) / cnt[:, None]
    std = jnp.sqrt(var + 1e-5)
    return out / std[batch] * w[None, :] + b[None, :]


def kernel(x, edge_index, batch, enc_W, enc_b, gat_W, att_src, att_dst, gat_b, tag_W, tag_b, gc_W_rel, gc_b_rel, gc_W_root, gn1_w, gn1_b, gn1_a, gn2_w, gn2_b, gn2_a, cls_W1, cls_b1, cls_W2, cls_b2):
    row = edge_index[0]
    col = edge_index[1]
    h, hp_flat, a_src, a_dst = _encoder(x, enc_W, enc_b, gat_W, att_src, att_dst)
    hp = hp_flat.reshape(N, HEADS, HD)
    e = jax.nn.leaky_relu(a_src[row] + a_dst[col], negative_slope=0.2)
    e_max = jax.ops.segment_max(e, col, num_segments=N)
    e_max = jnp.where(jnp.isfinite(e_max), e_max, 0.0)
    e_exp = jnp.exp(e - e_max[col])
    denom = jax.ops.segment_sum(e_exp, col, num_segments=N)
    alpha = e_exp / (denom[col] + 1e-16)
    msg = hp[row] * alpha[:, :, None]
    h = jax.ops.segment_sum(msg, col, num_segments=N).reshape(N, H) + gat_b
    h = jax.nn.relu(_graph_norm(h, batch, gn1_w, gn1_b, gn1_a))
    deg = jax.ops.segment_sum(jnp.ones((E,), jnp.float32), col, num_segments=N)
    dinv = jnp.where(deg > 0, 1.0 / jnp.sqrt(jnp.maximum(deg, 1.0)), 0.0)
    norm = dinv[row] * dinv[col]
    xs = h
    out = xs @ tag_W[0]
    for k in range(1, K + 1):
        xs = jax.ops.segment_sum(norm[:, None] * xs[row], col, num_segments=N)
        out = out + xs @ tag_W[k]
    h = out + tag_b
    h = jax.nn.relu(_graph_norm(h, batch, gn2_w, gn2_b, gn2_a))
    agg = jax.ops.segment_sum(h[row], col, num_segments=N)
    h = jax.nn.relu(agg @ gc_W_rel + gc_b_rel + h @ gc_W_root)
    x_max = jax.ops.segment_max(h, batch, num_segments=G)
    x_max = jnp.where(jnp.isfinite(x_max), x_max, 0.0)
    cnt = jnp.maximum(jax.ops.segment_sum(jnp.ones((N,), jnp.float32), batch, num_segments=G), 1.0)
    x_mean = jax.ops.segment_sum(h, batch, num_segments=G) / cnt[:, None]
    xp = jnp.concatenate([x_max, x_mean], axis=1)
    hc = jax.nn.gelu(xp @ cls_W1 + cls_b1, approximate=False)
    logits = hc @ cls_W2 + cls_b2
    return jax.nn.log_softmax(logits, axis=-1)
